# baseline (speedup 1.0000x reference)
import math

import jax
import jax.numpy as jnp
from jax import lax
from jax.experimental import pallas as pl
from jax.experimental.pallas import tpu as pltpu
from jax.experimental.pallas import tpu_sc as plsc

VOCAB = 100000
CTX = 2048
DIM = 768
BATCH = 4
SCALE = math.sqrt(DIM)

ROWS = BATCH * CTX
NW = 32
PPW = CTX // NW
PC = 8
NCHUNK = PPW // PC
NBUF = 2
LANES = 16
VPR = DIM // LANES


def _emb_body(x_hbm, tab_hbm, pos_hbm, out_hbm, idx_v, rows_v, pos_v,
              gsem0, gsem1, ssem0, ssem1):
    gsems = (gsem0, gsem1)
    ssems = (ssem0, ssem1)
    cid = lax.axis_index("c")
    sid = lax.axis_index("s")
    wid = sid * 2 + cid
    pbase = wid * PPW

    pltpu.sync_copy(pos_hbm.at[pl.ds(pbase, PPW)], pos_v)
    for b in range(BATCH):
        pltpu.sync_copy(x_hbm.at[pl.ds(b * CTX + pbase, PPW)], idx_v.at[b])

    def issue(k, bsel):
        for b in range(BATCH):
            pltpu.async_copy(
                tab_hbm.at[idx_v.at[b, pl.ds(k * PC, PC)]],
                rows_v.at[bsel, b], gsems[bsel])

    def wait_gathers(bsel):
        for b in range(BATCH):
            pltpu.make_async_copy(
                tab_hbm.at[idx_v.at[b, pl.ds(0, PC)]],
                rows_v.at[bsel, b], gsems[bsel]).wait()

    def wait_stores(bsel):
        for b in range(BATCH):
            pltpu.make_async_copy(
                rows_v.at[bsel, b],
                out_hbm.at[pl.ds(b * CTX + pbase, PC)], ssems[bsel]).wait()

    def compute_and_store(k, bsel):
        buf = rows_v.at[bsel]

        @plsc.parallel_loop(0, VPR, 1, unroll=2)
        def _col_body(j):
            sl = pl.ds(j * LANES, LANES)
            for p in range(PC):
                pv = pos_v[k * PC + p, sl]
                for b in range(BATCH):
                    buf[b, p, sl] = buf[b, p, sl] * SCALE + pv

        for b in range(BATCH):
            pltpu.async_copy(
                buf.at[b],
                out_hbm.at[pl.ds(b * CTX + pbase + k * PC, PC)], ssems[bsel])

    for d in range(NBUF):
        issue(d, d)

    @pl.loop(0, NCHUNK - NBUF, step=NBUF)
    def _chunk_loop(k):
        for d in range(NBUF):
            wait_gathers(d)
            compute_and_store(k + d, d)
            wait_stores(d)
            issue(k + d + NBUF, d)

    for k in range(NCHUNK - NBUF, NCHUNK):
        d = k % NBUF
        wait_gathers(d)
        compute_and_store(k, d)
        wait_stores(d)


def kernel(x, token_emb, pos_enc):
    x_flat = x.reshape(ROWS).astype(jnp.int32)
    pos2d = pos_enc.reshape(CTX, DIM)

    mesh = plsc.VectorSubcoreMesh(core_axis_name="c", subcore_axis_name="s")
    out = pl.kernel(
        _emb_body,
        mesh=mesh,
        out_type=jax.ShapeDtypeStruct((ROWS, DIM), jnp.float32),
        scratch_types=[
            pltpu.VMEM((BATCH, PPW), jnp.int32),
            pltpu.VMEM((NBUF, BATCH, PC, DIM), jnp.float32),
            pltpu.VMEM((PPW, DIM), jnp.float32),
            pltpu.SemaphoreType.DMA,
            pltpu.SemaphoreType.DMA,
            pltpu.SemaphoreType.DMA,
            pltpu.SemaphoreType.DMA,
        ],
    )(x_flat, token_emb, pos2d)
    return out.reshape(BATCH, CTX, DIM)


# speedup vs baseline: 1.2952x; 1.2952x over previous
import math

import jax
import jax.numpy as jnp
from jax import lax
from jax.experimental import pallas as pl
from jax.experimental.pallas import tpu as pltpu
from jax.experimental.pallas import tpu_sc as plsc

VOCAB = 100000
CTX = 2048
DIM = 768
BATCH = 4
SCALE = math.sqrt(DIM)

ROWS = BATCH * CTX
NW = 32
PPW = CTX // NW
PC = 8
NCHUNK = PPW // PC
NBUF = 3
LANES = 16
VPR = DIM // LANES


def _emb_body(x_hbm, tab_hbm, pos_hbm, out_hbm, idx_v, rows_v, pos_v,
              gsem0, gsem1, gsem2, ssem0, ssem1, ssem2):
    gsems = (gsem0, gsem1, gsem2)
    ssems = (ssem0, ssem1, ssem2)
    cid = lax.axis_index("c")
    sid = lax.axis_index("s")
    wid = sid * 2 + cid
    pbase = wid * PPW

    pltpu.sync_copy(pos_hbm.at[pl.ds(pbase, PPW)], pos_v)
    for b in range(BATCH):
        pltpu.sync_copy(x_hbm.at[pl.ds(b * CTX + pbase, PPW)], idx_v.at[b])

    def issue(k):
        bsel = k % NBUF
        return [
            pltpu.async_copy(
                tab_hbm.at[idx_v.at[b, pl.ds(k * PC, PC)]],
                rows_v.at[bsel, b], gsems[bsel])
            for b in range(BATCH)
        ]

    inflight = [None] * NCHUNK
    stores = [None] * NCHUNK
    for d in range(NBUF):
        inflight[d] = issue(d)
    for k in range(NCHUNK):
        bsel = k % NBUF
        for g in inflight[k]:
            g.wait()
        buf = rows_v.at[bsel]

        @plsc.parallel_loop(0, VPR, 1, unroll=1)
        def _col_body(j):
            sl = pl.ds(j * LANES, LANES)
            for p in range(PC):
                pv = pos_v[k * PC + p, sl]
                for b in range(BATCH):
                    buf[b, p, sl] = buf[b, p, sl] * SCALE + pv

        stores[k] = [
            pltpu.async_copy(
                buf.at[b],
                out_hbm.at[pl.ds(b * CTX + pbase + k * PC, PC)], ssems[bsel])
            for b in range(BATCH)
        ]
        if k + NBUF < NCHUNK:
            for s in stores[k]:
                s.wait()
            inflight[k + NBUF] = issue(k + NBUF)
    for k in range(max(NCHUNK - NBUF, 0), NCHUNK):
        for s in stores[k]:
            s.wait()


def kernel(x, token_emb, pos_enc):
    x_flat = x.reshape(ROWS).astype(jnp.int32)
    pos2d = pos_enc.reshape(CTX, DIM)

    mesh = plsc.VectorSubcoreMesh(core_axis_name="c", subcore_axis_name="s")
    out = pl.kernel(
        _emb_body,
        mesh=mesh,
        out_type=jax.ShapeDtypeStruct((ROWS, DIM), jnp.float32),
        scratch_types=[
            pltpu.VMEM((BATCH, PPW), jnp.int32),
            pltpu.VMEM((NBUF, BATCH, PC, DIM), jnp.float32),
            pltpu.VMEM((PPW, DIM), jnp.float32),
            pltpu.SemaphoreType.DMA,
            pltpu.SemaphoreType.DMA,
            pltpu.SemaphoreType.DMA,
            pltpu.SemaphoreType.DMA,
            pltpu.SemaphoreType.DMA,
            pltpu.SemaphoreType.DMA,
        ],
    )(x_flat, token_emb, pos2d)
    return out.reshape(BATCH, CTX, DIM)


# speedup vs baseline: 1.3753x; 1.0618x over previous
import math

import jax
import jax.numpy as jnp
from jax import lax
from jax.experimental import pallas as pl
from jax.experimental.pallas import tpu as pltpu
from jax.experimental.pallas import tpu_sc as plsc

VOCAB = 100000
CTX = 2048
DIM = 768
BATCH = 4
SCALE = math.sqrt(DIM)

ROWS = BATCH * CTX
NW = 32
PPW = CTX // NW
PC = 8
NCHUNK = PPW // PC
NBUF = 3
LANES = 16
VPR = DIM // LANES


def _emb_body(x_hbm, tab_hbm, pos_hbm, out_hbm, idx_v, rows_v, pos_v,
              psem, gsem0, gsem1, gsem2, ssem0, ssem1, ssem2):
    gsems = (gsem0, gsem1, gsem2)
    ssems = (ssem0, ssem1, ssem2)
    cid = lax.axis_index("c")
    sid = lax.axis_index("s")
    wid = sid * 2 + cid
    pbase = wid * PPW

    pos_cp = pltpu.async_copy(pos_hbm.at[pl.ds(pbase, PPW)], pos_v, psem)
    pltpu.sync_copy(x_hbm.at[wid], idx_v)

    def issue(k):
        bsel = k % NBUF
        return pltpu.async_copy(
            tab_hbm.at[idx_v.at[k]], rows_v.at[bsel], gsems[bsel])

    inflight = [None] * NCHUNK
    stores = [None] * NCHUNK
    for d in range(NBUF):
        inflight[d] = issue(d)
    pos_cp.wait()
    for k in range(NCHUNK):
        bsel = k % NBUF
        inflight[k].wait()
        buf = rows_v.at[bsel]

        @plsc.parallel_loop(0, VPR, 1, unroll=1)
        def _col_body(j):
            sl = pl.ds(j * LANES, LANES)
            for p in range(PC):
                pv = pos_v[k * PC + p, sl]
                for b in range(BATCH):
                    buf[b * PC + p, sl] = buf[b * PC + p, sl] * SCALE + pv

        stores[k] = [
            pltpu.async_copy(
                buf.at[pl.ds(b * PC, PC)],
                out_hbm.at[pl.ds(b * CTX + pbase + k * PC, PC)], ssems[bsel])
            for b in range(BATCH)
        ]
        if k + NBUF < NCHUNK:
            for s in stores[k]:
                s.wait()
            inflight[k + NBUF] = issue(k + NBUF)
    for k in range(max(NCHUNK - NBUF, 0), NCHUNK):
        for s in stores[k]:
            s.wait()


def kernel(x, token_emb, pos_enc):
    x_cm = (x.astype(jnp.int32)
            .reshape(BATCH, NW, NCHUNK, PC)
            .transpose(1, 2, 0, 3)
            .reshape(NW, NCHUNK, BATCH * PC))
    pos2d = pos_enc.reshape(CTX, DIM)

    mesh = plsc.VectorSubcoreMesh(core_axis_name="c", subcore_axis_name="s")
    out = pl.kernel(
        _emb_body,
        mesh=mesh,
        out_type=jax.ShapeDtypeStruct((ROWS, DIM), jnp.float32),
        scratch_types=[
            pltpu.VMEM((NCHUNK, BATCH * PC), jnp.int32),
            pltpu.VMEM((NBUF, BATCH * PC, DIM), jnp.float32),
            pltpu.VMEM((PPW, DIM), jnp.float32),
            pltpu.SemaphoreType.DMA,
            pltpu.SemaphoreType.DMA,
            pltpu.SemaphoreType.DMA,
            pltpu.SemaphoreType.DMA,
            pltpu.SemaphoreType.DMA,
            pltpu.SemaphoreType.DMA,
            pltpu.SemaphoreType.DMA,
        ],
    )(x_cm, token_emb, pos2d)
    return out.reshape(BATCH, CTX, DIM)
